# baseline (speedup 1.0000x reference)
import functools

import jax
import jax.numpy as jnp
from jax import lax
from jax.experimental import pallas as pl
from jax.experimental.pallas import tpu as pltpu
from jax.experimental.pallas import tpu_sc as plsc

_L = 16
_D = 128


def _make_sc_gather(num_img, vocab, row_pad):
    info = plsc.get_sparse_core_info()
    nc, ns = info.num_cores, info.num_subcores
    nw = nc * ns
    assert num_img % nw == 0
    rpt = num_img // nw
    ng = row_pad // _L

    mesh = plsc.VectorSubcoreMesh(core_axis_name="c", subcore_axis_name="s")

    @functools.partial(
        pl.kernel,
        out_type=jax.ShapeDtypeStruct((num_img, row_pad), jnp.float32),
        mesh=mesh,
        compiler_params=pltpu.CompilerParams(needs_layout_passes=False),
        scratch_types=[
            pltpu.VMEM((rpt, row_pad), jnp.int32),
            pltpu.VMEM((rpt, row_pad), jnp.int32),
            pltpu.VMEM((rpt, row_pad, _D), jnp.float32),
            pltpu.VMEM((rpt, row_pad), jnp.float32),
            pltpu.SemaphoreType.DMA,
        ],
    )
    def sc_kernel(in2d_hbm, rows_hbm, cols_hbm, out_hbm,
                  rows_v, cols_v, gath_v, vals_v, sem):
        cid = lax.axis_index("c")
        sid = lax.axis_index("s")
        wid = sid * nc + cid
        base = wid * rpt

        pltpu.sync_copy(rows_hbm.at[pl.ds(base, rpt)], rows_v)
        pltpu.sync_copy(cols_hbm.at[pl.ds(base, rpt)], cols_v)

        copies = [
            pltpu.async_copy(in2d_hbm.at[rows_v.at[r]], gath_v.at[r], sem)
            for r in range(rpt)
        ]
        for c in copies:
            c.wait()

        iota = lax.iota(jnp.int32, _L)
        for r in range(rpt):
            for g in range(ng):
                pos = iota + g * _L
                col = cols_v[r, pl.ds(g * _L, _L)]
                vals_v[r, pl.ds(g * _L, _L)] = plsc.load_gather(
                    gath_v.at[r], [pos, col]
                )

        pltpu.sync_copy(vals_v, out_hbm.at[pl.ds(base, rpt)])

    return sc_kernel


def _make_tc_reduce(num_img, per_img, row_pad):
    def body(t_ref, v_ref, o_ref):
        t = t_ref[...]
        lane = lax.broadcasted_iota(jnp.int32, (num_img, row_pad), 1)
        dup = None
        for s in range(1, per_img):
            hit = (t == pltpu.roll(t, s, axis=1)) & (lane >= s)
            dup = hit if dup is None else dup | hit
        w = jnp.logical_not(dup) & (lane < per_img)

        vals = v_ref[...]
        uniq = jnp.sum(jnp.where(w, vals, 0.0))
        lens = jnp.sum(w.astype(jnp.float32), axis=1, keepdims=True)
        m = jnp.max(lens)
        in0 = vals[:, per_img : per_img + 1]
        corr = jnp.sum((m - lens) * in0)
        o_ref[...] = jnp.broadcast_to(
            -(uniq + corr) / (jnp.float32(num_img) * m), (1, 1)
        )

    return body


@jax.jit
def kernel(input, target):
    num_img, vocab = input.shape
    per_img = (target.shape[0] // num_img) * target.shape[1]
    row_pad = 128
    assert (num_img * vocab) % _D == 0
    tgt = target.reshape(num_img, per_img).astype(jnp.int32)

    img = lax.broadcasted_iota(jnp.int32, (num_img, 1), 0)
    flat = img * vocab + tgt
    flat_pad = jnp.broadcast_to(img * vocab, (num_img, row_pad - per_img))
    flat_full = jnp.concatenate([flat, flat_pad], axis=1)
    rows_full = flat_full // _D
    cols_full = flat_full % _D

    in2d = input.reshape(num_img * vocab // _D, _D)
    sc = _make_sc_gather(num_img, vocab, row_pad)
    vals = sc(in2d, rows_full, cols_full)

    pad = -1 - lax.broadcasted_iota(jnp.int32, (num_img, row_pad - per_img), 1)
    tpad = jnp.concatenate([tgt, pad], axis=1)

    out = pl.pallas_call(
        _make_tc_reduce(num_img, per_img, row_pad),
        out_shape=jax.ShapeDtypeStruct((1, 1), jnp.float32),
    )(tpad, vals)
    return out[0, 0]


# speedup vs baseline: 1.6279x; 1.6279x over previous
import functools

import jax
import jax.numpy as jnp
from jax import lax
from jax.experimental import pallas as pl
from jax.experimental.pallas import tpu as pltpu
from jax.experimental.pallas import tpu_sc as plsc

_L = 16


def _make_sc_gather(num_img, vocab, row_pad):
    info = plsc.get_sparse_core_info()
    nc, ns = info.num_cores, info.num_subcores
    nw = nc * ns
    assert num_img % nw == 0
    rpt = num_img // nw
    ng = row_pad // _L

    mesh = plsc.VectorSubcoreMesh(core_axis_name="c", subcore_axis_name="s")

    @functools.partial(
        pl.kernel,
        out_type=jax.ShapeDtypeStruct((num_img, row_pad), jnp.float32),
        mesh=mesh,
        compiler_params=pltpu.CompilerParams(needs_layout_passes=False),
        scratch_types=[
            pltpu.VMEM((rpt, row_pad), jnp.int32),
            pltpu.VMEM((vocab,), jnp.float32),
            pltpu.VMEM((rpt, row_pad), jnp.float32),
        ],
    )
    def sc_kernel(in_hbm, tgt_hbm, out_hbm, idx_v, row_v, vals_v):
        cid = lax.axis_index("c")
        sid = lax.axis_index("s")
        wid = sid * nc + cid
        base = wid * rpt

        pltpu.sync_copy(tgt_hbm.at[pl.ds(base, rpt)], idx_v)

        for r in range(rpt):
            pltpu.sync_copy(in_hbm.at[base + r], row_v)
            for g in range(ng):
                t16 = idx_v[r, pl.ds(g * _L, _L)]
                vals_v[r, pl.ds(g * _L, _L)] = plsc.load_gather(row_v, [t16])

        pltpu.sync_copy(vals_v, out_hbm.at[pl.ds(base, rpt)])

    return sc_kernel


def _make_tc_reduce(num_img, per_img, row_pad):
    def body(t_ref, v_ref, o_ref):
        t = t_ref[...]
        lane = lax.broadcasted_iota(jnp.int32, (num_img, row_pad), 1)
        dup = None
        for s in range(1, per_img):
            hit = (t == pltpu.roll(t, s, axis=1)) & (lane >= s)
            dup = hit if dup is None else dup | hit
        w = jnp.logical_not(dup) & (lane < per_img)

        vals = v_ref[...]
        uniq = jnp.sum(jnp.where(w, vals, 0.0))
        lens = jnp.sum(w.astype(jnp.float32), axis=1, keepdims=True)
        m = jnp.max(lens)
        in0 = vals[:, per_img : per_img + 1]
        corr = jnp.sum((m - lens) * in0)
        o_ref[...] = jnp.broadcast_to(
            -(uniq + corr) / (jnp.float32(num_img) * m), (1, 1)
        )

    return body


@jax.jit
def kernel(input, target):
    num_img, vocab = input.shape
    per_img = (target.shape[0] // num_img) * target.shape[1]
    row_pad = 128
    tgt = target.reshape(num_img, per_img).astype(jnp.int32)

    ids_full = jnp.concatenate(
        [tgt, jnp.zeros((num_img, row_pad - per_img), jnp.int32)], axis=1)

    sc = _make_sc_gather(num_img, vocab, row_pad)
    vals = sc(input, ids_full)

    pad = -1 - lax.broadcasted_iota(jnp.int32, (num_img, row_pad - per_img), 1)
    tpad = jnp.concatenate([tgt, pad], axis=1)

    out = pl.pallas_call(
        _make_tc_reduce(num_img, per_img, row_pad),
        out_shape=jax.ShapeDtypeStruct((1, 1), jnp.float32),
    )(tpad, vals)
    return out[0, 0]
